# baseline (speedup 1.0000x reference)
import functools

import jax
import jax.numpy as jnp
import numpy as np
from jax import lax
from jax.experimental import pallas as pl
from jax.experimental.pallas import tpu as pltpu
from jax.experimental.pallas import tpu_sc as plsc

_NC = 2
_NS = 16
_NW = _NC * _NS

_K = 32
_H = 128
_CH = 4
_CR = _CH * _K
_NB = 4


def _matvec_tc(x, v):
    rows, h = x.shape
    br = 1024
    g = -(-rows // br)

    def body(x_ref, v_ref, o_ref):
        x4 = x_ref[...].reshape(8, 128, h)
        a = v_ref[...].reshape(h)
        o_ref[...] = jnp.sum(x4 * a, axis=2)

    out = pl.pallas_call(
        body,
        grid=(g,),
        in_specs=[
            pl.BlockSpec((br, h), lambda i: (i, 0)),
            pl.BlockSpec((1, h), lambda i: (0, 0)),
        ],
        out_specs=pl.BlockSpec((8, 128), lambda i: (i, 0)),
        out_shape=jax.ShapeDtypeStruct((8 * g, 128), jnp.float32),
    )(x, v.reshape(1, h))
    return out.reshape(br * g)


_DNUMS = lax.GatherDimensionNumbers(
    offset_dims=(), collapsed_slice_dims=(0,), start_index_map=(0,))


def _gather16(vec16, idx16):
    return lax.gather(vec16, idx16[:, None], _DNUMS, slice_sizes=(1,),
                      mode=lax.GatherScatterMode.PROMISE_IN_BOUNDS)


def _iota16():
    return lax.iota(jnp.int32, 16)


def _bcast_lane(vec16, i):
    return _gather16(vec16, _iota16() * 0 + i)


def _allmax16(v):
    iota = _iota16()
    for s in (8, 4, 2, 1):
        v = jnp.maximum(v, _gather16(v, iota ^ s))
    return v


def _allsum16(v):
    iota = _iota16()
    for s in (8, 4, 2, 1):
        v = v + _gather16(v, iota ^ s)
    return v


def _make_sc_kernel(pw):
    chunks = pw // _CH
    gmax = chunks // _NB - 1

    mesh = plsc.VectorSubcoreMesh(core_axis_name="c", subcore_axis_name="s")
    np_total = _NW * pw

    @functools.partial(
        pl.kernel,
        out_type=jax.ShapeDtypeStruct((np_total, _H), jnp.float32),
        mesh=mesh,
        scratch_types=[
            pltpu.VMEM((chunks, _CR), jnp.int32),
            pltpu.VMEM((chunks, 16), jnp.float32),
            pltpu.VMEM((_NB * _CR,), jnp.float32),
            pltpu.VMEM((_NB * _CR, _H), jnp.float32),
            pltpu.VMEM((pw, _H), jnp.float32),
            [pltpu.SemaphoreType.DMA] * _NB,
            [pltpu.SemaphoreType.DMA] * _NB,
        ],
    )
    def sc_kernel(neib_hbm, idx_hbm, sneib_hbm, spro_hbm, out_hbm,
                  idx_v, spro_v, sneib_v, rows_v, out_v,
                  sem_r, sem_s):
        wid = lax.axis_index("s") * _NC + lax.axis_index("c")
        pltpu.sync_copy(idx_hbm.at[wid], idx_v)
        pltpu.sync_copy(spro_hbm.at[wid], spro_v)

        def fire(c, b):
            pltpu.async_copy(neib_hbm.at[idx_v.at[c]],
                             rows_v.at[pl.ds(b * _CR, _CR)], sem_r[b])
            pltpu.async_copy(sneib_hbm.at[idx_v.at[c]],
                             sneib_v.at[pl.ds(b * _CR, _CR)], sem_s[b])

        def drain(c, b):
            pltpu.make_async_copy(neib_hbm.at[idx_v.at[c]],
                                  rows_v.at[pl.ds(b * _CR, _CR)],
                                  sem_r[b]).wait()
            pltpu.make_async_copy(sneib_hbm.at[idx_v.at[c]],
                                  sneib_v.at[pl.ds(b * _CR, _CR)],
                                  sem_s[b]).wait()

        for b in range(_NB):
            fire(b, b)

        def outer(g, carry):
            for b in range(_NB):
                c = _NB * g + b
                drain(c, b)

                def node_body(n, acc, b=b, c=c):
                    base = b * _CR + n * _K
                    node = c * _CH + n
                    sp = _bcast_lane(spro_v[c], n)
                    lo = sneib_v[pl.ds(base, 16)] + sp
                    hi = sneib_v[pl.ds(base + 16, 16)] + sp
                    lo = jnp.where(lo > 0.0, lo, lo * 0.01)
                    hi = jnp.where(hi > 0.0, hi, hi * 0.01)
                    m = _allmax16(jnp.maximum(lo, hi))
                    el = jnp.exp(lo - m)
                    eh = jnp.exp(hi - m)
                    inv = 1.0 / _allsum16(el + eh)
                    wl = el * inv
                    wh = eh * inv
                    accs = [jnp.zeros((16,), jnp.float32) for _ in range(8)]
                    for k in range(_K):
                        src = wl if k < 16 else wh
                        w = _bcast_lane(src, k % 16)
                        for j in range(8):
                            accs[j] = accs[j] + w * rows_v[base + k,
                                                           pl.ds(j * 16, 16)]
                    for j in range(8):
                        out_v[node, pl.ds(j * 16, 16)] = accs[j]
                    return acc

                lax.fori_loop(0, _CH, node_body, 0)

                @pl.when(g <= gmax - 1)
                def _prefetch(c=c, b=b):
                    fire(c + _NB, b)
            return carry

        lax.fori_loop(0, chunks // _NB, outer, 0)
        pltpu.sync_copy(out_v, out_hbm.at[pl.ds(wid * pw, pw)])

    return sc_kernel


def kernel(pro_feature, neibor_feature, now_neibor_list, attn):
    n, h = pro_feature.shape
    assert h == _H and now_neibor_list.shape[1] == _K

    s_pro = _matvec_tc(pro_feature, attn[:_H, 0])
    s_neib = _matvec_tc(neibor_feature, attn[_H:, 0])

    gran = _NW * _CH * _NB
    np_total = ((n + gran - 1) // gran) * gran
    pw = np_total // _NW
    chunks = pw // _CH

    m = neibor_feature.shape[0]
    fill = (jnp.arange(np_total * _K, dtype=jnp.int32) % m).reshape(
        np_total, _K)
    idx = now_neibor_list.astype(jnp.int32)
    idx_pad = fill.at[:n].set(idx)
    idx3 = idx_pad.reshape(_NW, chunks, _CR)
    spro_pad = jnp.zeros((np_total,), jnp.float32).at[:n].set(s_pro[:n])
    spro3 = jnp.pad(spro_pad.reshape(_NW, chunks, _CH),
                    ((0, 0), (0, 0), (0, 16 - _CH)))

    out = _make_sc_kernel(pw)(neibor_feature, idx3, s_neib, spro3)
    return out[:n]


# speedup vs baseline: 1.0001x; 1.0001x over previous
import functools

import jax
import jax.numpy as jnp
import numpy as np
from jax import lax
from jax.experimental import pallas as pl
from jax.experimental.pallas import tpu as pltpu
from jax.experimental.pallas import tpu_sc as plsc

_NC = 2
_NS = 16
_NW = _NC * _NS

_K = 32
_H = 128
_CH = 4
_CR = _CH * _K
_NB = 4


def _matvec_tc(x, v):
    rows, h = x.shape
    br = 1024
    g = -(-rows // br)

    def body(x_ref, v_ref, o_ref):
        x4 = x_ref[...].reshape(8, 128, h)
        a = v_ref[...].reshape(h)
        o_ref[...] = lax.dot_general(x4, a, (((2,), (0,)), ((), ())),
                                     preferred_element_type=jnp.float32)

    out = pl.pallas_call(
        body,
        grid=(g,),
        in_specs=[
            pl.BlockSpec((br, h), lambda i: (i, 0)),
            pl.BlockSpec((1, h), lambda i: (0, 0)),
        ],
        out_specs=pl.BlockSpec((8, 128), lambda i: (i, 0)),
        out_shape=jax.ShapeDtypeStruct((8 * g, 128), jnp.float32),
    )(x, v.reshape(1, h))
    return out.reshape(br * g)


_DNUMS = lax.GatherDimensionNumbers(
    offset_dims=(), collapsed_slice_dims=(0,), start_index_map=(0,))


def _gather16(vec16, idx16):
    return lax.gather(vec16, idx16[:, None], _DNUMS, slice_sizes=(1,),
                      mode=lax.GatherScatterMode.PROMISE_IN_BOUNDS)


def _iota16():
    return lax.iota(jnp.int32, 16)


def _bcast_lane(vec16, i):
    return _gather16(vec16, _iota16() * 0 + i)


def _allmax16(v):
    iota = _iota16()
    for s in (8, 4, 2, 1):
        v = jnp.maximum(v, _gather16(v, iota ^ s))
    return v


def _allsum16(v):
    iota = _iota16()
    for s in (8, 4, 2, 1):
        v = v + _gather16(v, iota ^ s)
    return v


def _make_sc_kernel(pw):
    chunks = pw // _CH
    gmax = chunks // _NB - 1

    mesh = plsc.VectorSubcoreMesh(core_axis_name="c", subcore_axis_name="s")
    np_total = _NW * pw

    @functools.partial(
        pl.kernel,
        out_type=jax.ShapeDtypeStruct((np_total, _H), jnp.float32),
        mesh=mesh,
        scratch_types=[
            pltpu.VMEM((chunks, _CR), jnp.int32),
            pltpu.VMEM((chunks, 16), jnp.float32),
            pltpu.VMEM((_NB * _CR,), jnp.float32),
            pltpu.VMEM((_NB * _CR, _H), jnp.float32),
            pltpu.VMEM((pw, _H), jnp.float32),
            [pltpu.SemaphoreType.DMA] * _NB,
            [pltpu.SemaphoreType.DMA] * _NB,
        ],
    )
    def sc_kernel(neib_hbm, idx_hbm, sneib_hbm, spro_hbm, out_hbm,
                  idx_v, spro_v, sneib_v, rows_v, out_v,
                  sem_r, sem_s):
        wid = lax.axis_index("s") * _NC + lax.axis_index("c")
        pltpu.sync_copy(idx_hbm.at[wid], idx_v)
        pltpu.sync_copy(spro_hbm.at[wid], spro_v)

        def fire(c, b):
            pltpu.async_copy(neib_hbm.at[idx_v.at[c]],
                             rows_v.at[pl.ds(b * _CR, _CR)], sem_r[b])
            pltpu.async_copy(sneib_hbm.at[idx_v.at[c]],
                             sneib_v.at[pl.ds(b * _CR, _CR)], sem_s[b])

        def drain(c, b):
            pltpu.make_async_copy(neib_hbm.at[idx_v.at[c]],
                                  rows_v.at[pl.ds(b * _CR, _CR)],
                                  sem_r[b]).wait()
            pltpu.make_async_copy(sneib_hbm.at[idx_v.at[c]],
                                  sneib_v.at[pl.ds(b * _CR, _CR)],
                                  sem_s[b]).wait()

        for b in range(_NB):
            fire(b, b)

        def outer(g, carry):
            for b in range(_NB):
                c = _NB * g + b
                drain(c, b)

                def node_body(n, acc, b=b, c=c):
                    base = b * _CR + n * _K
                    node = c * _CH + n
                    sp = _bcast_lane(spro_v[c], n)
                    lo = sneib_v[pl.ds(base, 16)] + sp
                    hi = sneib_v[pl.ds(base + 16, 16)] + sp
                    lo = jnp.where(lo > 0.0, lo, lo * 0.01)
                    hi = jnp.where(hi > 0.0, hi, hi * 0.01)
                    m = _allmax16(jnp.maximum(lo, hi))
                    el = jnp.exp(lo - m)
                    eh = jnp.exp(hi - m)
                    inv = 1.0 / _allsum16(el + eh)
                    wl = el * inv
                    wh = eh * inv
                    accs = [jnp.zeros((16,), jnp.float32) for _ in range(8)]
                    for k in range(_K):
                        src = wl if k < 16 else wh
                        w = _bcast_lane(src, k % 16)
                        for j in range(8):
                            accs[j] = accs[j] + w * rows_v[base + k,
                                                           pl.ds(j * 16, 16)]
                    for j in range(8):
                        out_v[node, pl.ds(j * 16, 16)] = accs[j]
                    return acc

                lax.fori_loop(0, _CH, node_body, 0)

                @pl.when(g <= gmax - 1)
                def _prefetch(c=c, b=b):
                    fire(c + _NB, b)
            return carry

        lax.fori_loop(0, chunks // _NB, outer, 0)
        pltpu.sync_copy(out_v, out_hbm.at[pl.ds(wid * pw, pw)])

    return sc_kernel


def kernel(pro_feature, neibor_feature, now_neibor_list, attn):
    n, h = pro_feature.shape
    assert h == _H and now_neibor_list.shape[1] == _K

    s_pro = _matvec_tc(pro_feature, attn[:_H, 0])
    s_neib = _matvec_tc(neibor_feature, attn[_H:, 0])

    gran = _NW * _CH * _NB
    np_total = ((n + gran - 1) // gran) * gran
    pw = np_total // _NW
    chunks = pw // _CH

    m = neibor_feature.shape[0]
    fill = (jnp.arange(np_total * _K, dtype=jnp.int32) % m).reshape(
        np_total, _K)
    idx = now_neibor_list.astype(jnp.int32)
    idx_pad = fill.at[:n].set(idx)
    idx3 = idx_pad.reshape(_NW, chunks, _CR)
    spro_pad = jnp.zeros((np_total,), jnp.float32).at[:n].set(s_pro[:n])
    spro3 = jnp.pad(spro_pad.reshape(_NW, chunks, _CH),
                    ((0, 0), (0, 0), (0, 16 - _CH)))

    out = _make_sc_kernel(pw)(neibor_feature, idx3, s_neib, spro3)
    return out[:n]


# speedup vs baseline: 1.2345x; 1.2343x over previous
import functools

import jax
import jax.numpy as jnp
import numpy as np
from jax import lax
from jax.experimental import pallas as pl
from jax.experimental.pallas import tpu as pltpu
from jax.experimental.pallas import tpu_sc as plsc

_NC = 2
_NS = 16
_NW = _NC * _NS

_K = 32
_H = 128
_CH = 4
_CR = _CH * _K
_NB = 4


def _matvec_tc(x, v):
    rows, h = x.shape
    br = 4096
    g = -(-rows // br)

    def body(x_ref, v_ref, o_ref):
        x4 = x_ref[...].reshape(br // 128, 128, h)
        a = v_ref[...].reshape(h)
        o_ref[...] = lax.dot_general(x4, a, (((2,), (0,)), ((), ())),
                                     preferred_element_type=jnp.float32)

    out = pl.pallas_call(
        body,
        grid=(g,),
        in_specs=[
            pl.BlockSpec((br, h), lambda i: (i, 0)),
            pl.BlockSpec((1, h), lambda i: (0, 0)),
        ],
        out_specs=pl.BlockSpec((br // 128, 128), lambda i: (i, 0)),
        out_shape=jax.ShapeDtypeStruct((br // 128 * g, 128), jnp.float32),
    )(x, v.reshape(1, h))
    return out.reshape(br * g)


_DNUMS = lax.GatherDimensionNumbers(
    offset_dims=(), collapsed_slice_dims=(0,), start_index_map=(0,))


def _gather16(vec16, idx16):
    return lax.gather(vec16, idx16[:, None], _DNUMS, slice_sizes=(1,),
                      mode=lax.GatherScatterMode.PROMISE_IN_BOUNDS)


def _iota16():
    return lax.iota(jnp.int32, 16)


def _bcast_lane(vec16, i):
    return _gather16(vec16, _iota16() * 0 + i)


def _allmax16(v):
    iota = _iota16()
    for s in (8, 4, 2, 1):
        v = jnp.maximum(v, _gather16(v, iota ^ s))
    return v


def _allsum16(v):
    iota = _iota16()
    for s in (8, 4, 2, 1):
        v = v + _gather16(v, iota ^ s)
    return v


def _make_sc_kernel(pw):
    chunks = pw // _CH
    gmax = chunks // _NB - 1

    mesh = plsc.VectorSubcoreMesh(core_axis_name="c", subcore_axis_name="s")
    np_total = _NW * pw

    @functools.partial(
        pl.kernel,
        out_type=jax.ShapeDtypeStruct((np_total, _H), jnp.float32),
        mesh=mesh,
        scratch_types=[
            pltpu.VMEM((chunks, _CR), jnp.int32),
            pltpu.VMEM((chunks, 16), jnp.float32),
            pltpu.VMEM((_NB * _CR,), jnp.float32),
            pltpu.VMEM((_NB * _CR, _H), jnp.float32),
            pltpu.VMEM((pw, _H), jnp.float32),
            [pltpu.SemaphoreType.DMA] * _NB,
            [pltpu.SemaphoreType.DMA] * _NB,
        ],
    )
    def sc_kernel(neib_hbm, idx_hbm, sneib_hbm, spro_hbm, out_hbm,
                  idx_v, spro_v, sneib_v, rows_v, out_v,
                  sem_r, sem_s):
        wid = lax.axis_index("s") * _NC + lax.axis_index("c")
        pltpu.sync_copy(idx_hbm.at[wid], idx_v)
        pltpu.sync_copy(spro_hbm.at[wid], spro_v)

        def fire(c, b):
            pltpu.async_copy(neib_hbm.at[idx_v.at[c]],
                             rows_v.at[pl.ds(b * _CR, _CR)], sem_r[b])
            pltpu.async_copy(sneib_hbm.at[idx_v.at[c]],
                             sneib_v.at[pl.ds(b * _CR, _CR)], sem_s[b])

        def drain(c, b):
            pltpu.make_async_copy(neib_hbm.at[idx_v.at[c]],
                                  rows_v.at[pl.ds(b * _CR, _CR)],
                                  sem_r[b]).wait()
            pltpu.make_async_copy(sneib_hbm.at[idx_v.at[c]],
                                  sneib_v.at[pl.ds(b * _CR, _CR)],
                                  sem_s[b]).wait()

        for b in range(_NB):
            fire(b, b)

        def outer(g, carry):
            for b in range(_NB):
                c = _NB * g + b
                drain(c, b)

                def node_body(n, acc, b=b, c=c):
                    base = b * _CR + n * _K
                    node = c * _CH + n
                    sp = _bcast_lane(spro_v[c], n)
                    lo = sneib_v[pl.ds(base, 16)] + sp
                    hi = sneib_v[pl.ds(base + 16, 16)] + sp
                    lo = jnp.where(lo > 0.0, lo, lo * 0.01)
                    hi = jnp.where(hi > 0.0, hi, hi * 0.01)
                    m = _allmax16(jnp.maximum(lo, hi))
                    el = jnp.exp(lo - m)
                    eh = jnp.exp(hi - m)
                    inv = 1.0 / _allsum16(el + eh)
                    wl = el * inv
                    wh = eh * inv
                    accs = [jnp.zeros((16,), jnp.float32) for _ in range(8)]
                    for k in range(_K):
                        src = wl if k < 16 else wh
                        w = _bcast_lane(src, k % 16)
                        for j in range(8):
                            accs[j] = accs[j] + w * rows_v[base + k,
                                                           pl.ds(j * 16, 16)]
                    for j in range(8):
                        out_v[node, pl.ds(j * 16, 16)] = accs[j]
                    return acc

                lax.fori_loop(0, _CH, node_body, 0)

                @pl.when(g <= gmax - 1)
                def _prefetch(c=c, b=b):
                    fire(c + _NB, b)
            return carry

        lax.fori_loop(0, chunks // _NB, outer, 0)
        pltpu.sync_copy(out_v, out_hbm.at[pl.ds(wid * pw, pw)])

    return sc_kernel


def kernel(pro_feature, neibor_feature, now_neibor_list, attn):
    n, h = pro_feature.shape
    assert h == _H and now_neibor_list.shape[1] == _K

    s_pro = _matvec_tc(pro_feature, attn[:_H, 0])
    s_neib = _matvec_tc(neibor_feature, attn[_H:, 0])

    gran = _NW * _CH * _NB
    np_total = ((n + gran - 1) // gran) * gran
    pw = np_total // _NW
    chunks = pw // _CH

    m = neibor_feature.shape[0]
    fill = (jnp.arange(np_total * _K, dtype=jnp.int32) % m).reshape(
        np_total, _K)
    idx = now_neibor_list.astype(jnp.int32)
    idx_pad = fill.at[:n].set(idx)
    idx3 = idx_pad.reshape(_NW, chunks, _CR)
    spro_pad = jnp.zeros((np_total,), jnp.float32).at[:n].set(s_pro[:n])
    spro3 = jnp.pad(spro_pad.reshape(_NW, chunks, _CH),
                    ((0, 0), (0, 0), (0, 16 - _CH)))

    out = _make_sc_kernel(pw)(neibor_feature, idx3, s_neib, spro3)
    return out[:n]


# speedup vs baseline: 1.2834x; 1.0396x over previous
import functools

import jax
import jax.numpy as jnp
import numpy as np
from jax import lax
from jax.experimental import pallas as pl
from jax.experimental.pallas import tpu as pltpu
from jax.experimental.pallas import tpu_sc as plsc

_NC = 2
_NS = 16
_NW = _NC * _NS

_K = 32
_H = 128
_CH = 4
_CR = _CH * _K
_NB = 4


def _matvec_tc(x, v):
    rows, h = x.shape
    br = 8192
    g = -(-rows // br)

    def body(x_ref, v_ref, o_ref):
        x4 = x_ref[...].reshape(br // 128, 128, h)
        a = v_ref[...].reshape(h)
        o_ref[...] = lax.dot_general(x4, a, (((2,), (0,)), ((), ())),
                                     preferred_element_type=jnp.float32)

    out = pl.pallas_call(
        body,
        grid=(g,),
        in_specs=[
            pl.BlockSpec((br, h), lambda i: (i, 0)),
            pl.BlockSpec((1, h), lambda i: (0, 0)),
        ],
        out_specs=pl.BlockSpec((br // 128, 128), lambda i: (i, 0)),
        out_shape=jax.ShapeDtypeStruct((br // 128 * g, 128), jnp.float32),
    )(x, v.reshape(1, h))
    return out.reshape(br * g)


_DNUMS = lax.GatherDimensionNumbers(
    offset_dims=(), collapsed_slice_dims=(0,), start_index_map=(0,))


def _gather16(vec16, idx16):
    return lax.gather(vec16, idx16[:, None], _DNUMS, slice_sizes=(1,),
                      mode=lax.GatherScatterMode.PROMISE_IN_BOUNDS)


def _iota16():
    return lax.iota(jnp.int32, 16)


def _bcast_lane(vec16, i):
    return _gather16(vec16, _iota16() * 0 + i)


def _allmax16(v):
    iota = _iota16()
    for s in (8, 4, 2, 1):
        v = jnp.maximum(v, _gather16(v, iota ^ s))
    return v


def _allsum16(v):
    iota = _iota16()
    for s in (8, 4, 2, 1):
        v = v + _gather16(v, iota ^ s)
    return v


def _make_sc_kernel(pw):
    chunks = pw // _CH
    gmax = chunks // _NB - 1

    mesh = plsc.VectorSubcoreMesh(core_axis_name="c", subcore_axis_name="s")
    np_total = _NW * pw

    @functools.partial(
        pl.kernel,
        out_type=jax.ShapeDtypeStruct((np_total, _H), jnp.float32),
        mesh=mesh,
        scratch_types=[
            pltpu.VMEM((chunks, _CR), jnp.int32),
            pltpu.VMEM((chunks, 16), jnp.float32),
            pltpu.VMEM((_NB * _CR,), jnp.float32),
            pltpu.VMEM((_NB * _CR, _H), jnp.float32),
            pltpu.VMEM((pw, _H), jnp.float32),
            [pltpu.SemaphoreType.DMA] * _NB,
            [pltpu.SemaphoreType.DMA] * _NB,
        ],
    )
    def sc_kernel(neib_hbm, idx_hbm, sneib_hbm, spro_hbm, out_hbm,
                  idx_v, spro_v, sneib_v, rows_v, out_v,
                  sem_r, sem_s):
        wid = lax.axis_index("s") * _NC + lax.axis_index("c")
        pltpu.sync_copy(idx_hbm.at[wid], idx_v)
        pltpu.sync_copy(spro_hbm.at[wid], spro_v)

        def fire(c, b):
            pltpu.async_copy(neib_hbm.at[idx_v.at[c]],
                             rows_v.at[pl.ds(b * _CR, _CR)], sem_r[b])
            pltpu.async_copy(sneib_hbm.at[idx_v.at[c]],
                             sneib_v.at[pl.ds(b * _CR, _CR)], sem_s[b])

        def drain(c, b):
            pltpu.make_async_copy(neib_hbm.at[idx_v.at[c]],
                                  rows_v.at[pl.ds(b * _CR, _CR)],
                                  sem_r[b]).wait()
            pltpu.make_async_copy(sneib_hbm.at[idx_v.at[c]],
                                  sneib_v.at[pl.ds(b * _CR, _CR)],
                                  sem_s[b]).wait()

        for b in range(_NB):
            fire(b, b)

        def outer(g, carry):
            for b in range(_NB):
                c = _NB * g + b
                drain(c, b)

                def node_body(n, acc, b=b, c=c):
                    base = b * _CR + n * _K
                    node = c * _CH + n
                    sp = _bcast_lane(spro_v[c], n)
                    lo = sneib_v[pl.ds(base, 16)] + sp
                    hi = sneib_v[pl.ds(base + 16, 16)] + sp
                    lo = jnp.where(lo > 0.0, lo, lo * 0.01)
                    hi = jnp.where(hi > 0.0, hi, hi * 0.01)
                    m = _allmax16(jnp.maximum(lo, hi))
                    el = jnp.exp(lo - m)
                    eh = jnp.exp(hi - m)
                    inv = 1.0 / _allsum16(el + eh)
                    wl = el * inv
                    wh = eh * inv
                    accs = [jnp.zeros((16,), jnp.float32) for _ in range(8)]
                    for k in range(_K):
                        src = wl if k < 16 else wh
                        w = _bcast_lane(src, k % 16)
                        for j in range(8):
                            accs[j] = accs[j] + w * rows_v[base + k,
                                                           pl.ds(j * 16, 16)]
                    for j in range(8):
                        out_v[node, pl.ds(j * 16, 16)] = accs[j]
                    return acc

                lax.fori_loop(0, _CH, node_body, 0)

                @pl.when(g <= gmax - 1)
                def _prefetch(c=c, b=b):
                    fire(c + _NB, b)
            return carry

        lax.fori_loop(0, chunks // _NB, outer, 0)
        pltpu.sync_copy(out_v, out_hbm.at[pl.ds(wid * pw, pw)])

    return sc_kernel


def kernel(pro_feature, neibor_feature, now_neibor_list, attn):
    n, h = pro_feature.shape
    assert h == _H and now_neibor_list.shape[1] == _K

    s_pro = _matvec_tc(pro_feature, attn[:_H, 0])
    s_neib = _matvec_tc(neibor_feature, attn[_H:, 0])

    gran = _NW * _CH * _NB
    np_total = ((n + gran - 1) // gran) * gran
    pw = np_total // _NW
    chunks = pw // _CH

    m = neibor_feature.shape[0]
    fill = (jnp.arange(np_total * _K, dtype=jnp.int32) % m).reshape(
        np_total, _K)
    idx = now_neibor_list.astype(jnp.int32)
    idx_pad = fill.at[:n].set(idx)
    idx3 = idx_pad.reshape(_NW, chunks, _CR)
    spro_pad = jnp.zeros((np_total,), jnp.float32).at[:n].set(s_pro[:n])
    spro3 = jnp.pad(spro_pad.reshape(_NW, chunks, _CH),
                    ((0, 0), (0, 0), (0, 16 - _CH)))

    out = _make_sc_kernel(pw)(neibor_feature, idx3, s_neib, spro3)
    return out[:n]
